# baseline (speedup 1.0000x reference)
import jax
import jax.numpy as jnp
from jax.experimental import pallas as pl
from jax.experimental.pallas import tpu as pltpu

_MIB = 1 << 20


def _gcn_kernel(x_ref, adj_ref, w1_ref, b1_ref, w2_ref, b2_ref, o_ref):
    N = adj_ref.shape[0]
    half = N // 2
    x = x_ref[...]
    w1 = w1_ref[...]
    w2 = w2_ref[...]

    def layer1(rows):
        ax = jnp.dot(adj_ref[rows, :], x, preferred_element_type=jnp.float32)
        h1 = jnp.dot(ax, w1, preferred_element_type=jnp.float32)
        h1 = jnp.maximum(h1 + b1_ref[...], 0.0)
        return jnp.dot(h1, w2, preferred_element_type=jnp.float32)

    s2t = layer1(pl.ds(0, half))
    s2b = layer1(pl.ds(half, half))

    def layer2(rows):
        out = (jnp.dot(adj_ref[rows, pl.ds(0, half)], s2t,
                       preferred_element_type=jnp.float32) +
               jnp.dot(adj_ref[rows, pl.ds(half, half)], s2b,
                       preferred_element_type=jnp.float32))
        o_ref[rows, :] = jnp.tanh(out + b2_ref[...]).astype(o_ref.dtype)

    layer2(pl.ds(0, half))
    layer2(pl.ds(half, half))


def kernel(x, adj, w1, b1, w2, b2):
    B, N, nfeat = x.shape
    nhid = w1.shape[1]
    nclass = w2.shape[1]

    b1_2d = b1.reshape(1, nhid)
    b2_2d = b2.reshape(1, nclass)

    wspec = lambda shape: pl.BlockSpec(shape, lambda b: (0,) * len(shape))
    return pl.pallas_call(
        _gcn_kernel,
        out_shape=jax.ShapeDtypeStruct((B, N, nclass), x.dtype),
        grid=(B,),
        in_specs=[
            pl.BlockSpec((None, N, nfeat), lambda b: (b, 0, 0)),
            pl.BlockSpec((None, N, N), lambda b: (b, 0, 0)),
            wspec((nfeat, nhid)),
            wspec((1, nhid)),
            wspec((nhid, nclass)),
            wspec((1, nclass)),
        ],
        out_specs=pl.BlockSpec((None, N, nclass), lambda b: (b, 0, 0)),
        compiler_params=pltpu.CompilerParams(
            dimension_semantics=("arbitrary",),
            vmem_limit_bytes=64 * _MIB,
        ),
    )(x, adj, w1, b1_2d, w2, b2_2d)
